# Initial kernel scaffold; baseline (speedup 1.0000x reference)
import jax
import jax.numpy as jnp
from jax.experimental import pallas as pl


def kernel(vectors, labels, row_indices, feat_indices):
    raise NotImplementedError("write your pallas kernel here")


# speedup vs baseline: 1.5799x; 1.5799x over previous
import functools

import jax
import jax.numpy as jnp
from jax import lax
from jax.experimental import pallas as pl
from jax.experimental.pallas import tpu as pltpu
from jax.experimental.pallas import tpu_sc as plsc

NUM_CORES = 2
NUM_SUBCORES = 16
LANES = 16
CHUNK = 128


@functools.partial(jax.jit, static_argnums=())
def _forest_gather(vectors, labels_i32, row_idx, feat_idx):
    T, S = row_idx.shape
    N, D = vectors.shape
    F = feat_idx.shape[1]
    NW = NUM_CORES * NUM_SUBCORES
    assert F == LANES, "feature subset must match SC lane count"
    assert (T * S) % NW == 0
    per_w = (T * S) // NW
    assert NW % T == 0
    w_per_tree = NW // T
    assert per_w % CHUNK == 0
    nch = per_w // CHUNK

    mesh = plsc.VectorSubcoreMesh(
        core_axis_name="c", subcore_axis_name="s",
        num_cores=NUM_CORES, num_subcores=NUM_SUBCORES)

    @functools.partial(
        pl.kernel,
        out_type=[
            jax.ShapeDtypeStruct((T, S, F), jnp.float32),
            jax.ShapeDtypeStruct((T, S), jnp.int32),
        ],
        mesh=mesh,
        compiler_params=pltpu.CompilerParams(
            use_tc_tiling_on_sc=False, needs_layout_passes=False),
        scratch_types=[
            pltpu.VMEM((per_w,), jnp.int32),
            pltpu.VMEM((F,), jnp.int32),
            pltpu.VMEM((2, CHUNK, D), jnp.float32),
            pltpu.VMEM((2, CHUNK), jnp.int32),
            pltpu.VMEM((2, CHUNK, F), jnp.float32),
            pltpu.SemaphoreType.DMA((2,)),
            pltpu.SemaphoreType.DMA((2,)),
            pltpu.SemaphoreType.DMA((2,)),
            pltpu.SemaphoreType.DMA((2,)),
        ],
    )
    def run(vec_hbm, lab_hbm, ri_hbm, fi_hbm, out_hbm, olab_hbm,
            idx_v, feat_v, rbuf, lbuf, obuf, sem_r, sem_l, sem_so, sem_sl):
        wid = lax.axis_index("s") * NUM_CORES + lax.axis_index("c")
        t = wid // w_per_tree
        base = (wid % w_per_tree) * per_w

        pltpu.sync_copy(ri_hbm.at[t, pl.ds(base, per_w)], idx_v)
        pltpu.sync_copy(fi_hbm.at[t], feat_v)
        feat = feat_v[:]

        def start_gathers(c):
            p = c % 2
            idxs = idx_v.at[pl.ds(c * CHUNK, CHUNK)]
            pltpu.async_copy(vec_hbm.at[idxs], rbuf.at[p], sem_r.at[p])
            pltpu.async_copy(lab_hbm.at[idxs], lbuf.at[p], sem_l.at[p])

        def wait_stores(c):
            p = c % 2
            off = base + c * CHUNK
            pltpu.make_async_copy(
                obuf.at[p], out_hbm.at[t, pl.ds(off, CHUNK), :], sem_so.at[p]
            ).wait()
            pltpu.make_async_copy(
                lbuf.at[p], olab_hbm.at[t, pl.ds(off, CHUNK)], sem_sl.at[p]
            ).wait()

        start_gathers(0)

        def chunk_body(c, carry):
            p = c % 2

            @pl.when(c >= 1)
            def _():
                wait_stores(c - 1)

            @pl.when(c + 1 < nch)
            def _():
                start_gathers(c + 1)

            idxs = idx_v.at[pl.ds(c * CHUNK, CHUNK)]
            pltpu.make_async_copy(vec_hbm.at[idxs], rbuf.at[p], sem_r.at[p]).wait()
            pltpu.make_async_copy(lab_hbm.at[idxs], lbuf.at[p], sem_l.at[p]).wait()

            p16 = jnp.full((LANES,), p, jnp.int32)

            def sub(r, carry2):
                r16 = jnp.full((LANES,), r, jnp.int32)
                obuf[p, r, :] = plsc.load_gather(rbuf, [p16, r16, feat])
                return carry2

            lax.fori_loop(0, CHUNK, sub, None)

            off = base + c * CHUNK
            pltpu.async_copy(
                obuf.at[p], out_hbm.at[t, pl.ds(off, CHUNK), :], sem_so.at[p])
            pltpu.async_copy(
                lbuf.at[p], olab_hbm.at[t, pl.ds(off, CHUNK)], sem_sl.at[p])
            return carry

        lax.fori_loop(0, nch, chunk_body, None)
        wait_stores(nch - 1)

    return run(vectors, labels_i32, row_idx, feat_idx)


def kernel(vectors, labels, row_indices, feat_indices):
    featured, lab = _forest_gather(
        vectors,
        labels.astype(jnp.int32),
        row_indices.astype(jnp.int32),
        feat_indices.astype(jnp.int32),
    )
    return featured, lab.astype(labels.dtype)


# speedup vs baseline: 2.4533x; 1.5529x over previous
import functools

import jax
import jax.numpy as jnp
from jax import lax
from jax.experimental import pallas as pl
from jax.experimental.pallas import tpu as pltpu
from jax.experimental.pallas import tpu_sc as plsc

NUM_CORES = 2
NUM_SUBCORES = 16
LANES = 16
CHUNK = 128


def _forest_gather(vectors, labels_i32, ri_flat, fi_flat, T, S, F):
    N, D = vectors.shape
    NW = NUM_CORES * NUM_SUBCORES
    assert F == LANES, "feature subset must match SC lane count"
    assert D == 2 * 128
    assert (T * S) % NW == 0 and (T * S) // NW % CHUNK == 0 and S % ((T * S) // NW) == 0
    per_w = (T * S) // NW
    nch = per_w // CHUNK

    mesh = plsc.VectorSubcoreMesh(
        core_axis_name="c", subcore_axis_name="s",
        num_cores=NUM_CORES, num_subcores=NUM_SUBCORES)

    @functools.partial(
        pl.kernel,
        out_type=[
            jax.ShapeDtypeStruct((T * S * F,), jnp.float32),
            jax.ShapeDtypeStruct((T * S,), jnp.int32),
        ],
        mesh=mesh,
        compiler_params=pltpu.CompilerParams(needs_layout_passes=False),
        scratch_types=[
            pltpu.VMEM((per_w,), jnp.int32),
            pltpu.VMEM((F,), jnp.int32),
            pltpu.VMEM((2, CHUNK, 256), jnp.float32),
            pltpu.VMEM((2, CHUNK), jnp.int32),
            pltpu.VMEM((2, CHUNK * F), jnp.float32),
            pltpu.SemaphoreType.DMA((2,)),
            pltpu.SemaphoreType.DMA((2,)),
            pltpu.SemaphoreType.DMA((2,)),
            pltpu.SemaphoreType.DMA((2,)),
        ],
    )
    def run(vec_hbm, lab_hbm, ri_hbm, fi_hbm, out_hbm, olab_hbm,
            idx_v, feat_v, rbuf, lbuf, obuf, sem_r, sem_l, sem_so, sem_sl):
        wid = lax.axis_index("s") * NUM_CORES + lax.axis_index("c")
        g0 = wid * per_w
        t = g0 // S

        pltpu.sync_copy(ri_hbm.at[pl.ds(g0, per_w)], idx_v)
        pltpu.sync_copy(fi_hbm.at[pl.ds(t * F, F)], feat_v)
        feat = feat_v[:]

        def start_gathers(c):
            p = c % 2
            idxs = idx_v.at[pl.ds(c * CHUNK, CHUNK)]
            pltpu.async_copy(vec_hbm.at[idxs], rbuf.at[p], sem_r.at[p])
            pltpu.async_copy(lab_hbm.at[idxs], lbuf.at[p], sem_l.at[p])

        def wait_stores(c):
            p = c % 2
            pltpu.make_async_copy(
                obuf.at[p],
                out_hbm.at[pl.ds((g0 + c * CHUNK) * F, CHUNK * F)],
                sem_so.at[p]).wait()
            pltpu.make_async_copy(
                lbuf.at[p],
                olab_hbm.at[pl.ds(g0 + c * CHUNK, CHUNK)],
                sem_sl.at[p]).wait()

        start_gathers(0)

        def chunk_body(c, carry):
            p = c % 2

            @pl.when(c >= 1)
            def _():
                wait_stores(c - 1)

            @pl.when(c + 1 < nch)
            def _():
                start_gathers(c + 1)

            idxs = idx_v.at[pl.ds(c * CHUNK, CHUNK)]
            pltpu.make_async_copy(vec_hbm.at[idxs], rbuf.at[p], sem_r.at[p]).wait()
            pltpu.make_async_copy(lab_hbm.at[idxs], lbuf.at[p], sem_l.at[p]).wait()

            p16 = jnp.full((LANES,), p, jnp.int32)

            def sub(r, carry2):
                r16 = jnp.full((LANES,), r, jnp.int32)
                obuf[p, pl.ds(r * F, F)] = plsc.load_gather(
                    rbuf, [p16, r16, feat])
                return carry2

            lax.fori_loop(0, CHUNK, sub, None)

            pltpu.async_copy(
                obuf.at[p],
                out_hbm.at[pl.ds((g0 + c * CHUNK) * F, CHUNK * F)],
                sem_so.at[p])
            pltpu.async_copy(
                lbuf.at[p],
                olab_hbm.at[pl.ds(g0 + c * CHUNK, CHUNK)],
                sem_sl.at[p])
            return carry

        lax.fori_loop(0, nch, chunk_body, None)
        wait_stores(nch - 1)

    return run(vectors, labels_i32, ri_flat, fi_flat)


def kernel(vectors, labels, row_indices, feat_indices):
    T, S = row_indices.shape
    F = feat_indices.shape[1]
    featured_flat, lab_flat = _forest_gather(
        vectors,
        labels.astype(jnp.int32),
        row_indices.reshape(-1).astype(jnp.int32),
        feat_indices.reshape(-1).astype(jnp.int32),
        T, S, F,
    )
    featured = featured_flat.reshape(T, S, F)
    lab = lab_flat.reshape(T, S).astype(labels.dtype)
    return featured, lab


# speedup vs baseline: 2.5187x; 1.0266x over previous
import functools

import jax
import jax.numpy as jnp
from jax import lax
from jax.experimental import pallas as pl
from jax.experimental.pallas import tpu as pltpu
from jax.experimental.pallas import tpu_sc as plsc

NUM_CORES = 2
NUM_SUBCORES = 16
LANES = 16
CHUNK = 128


def _forest_gather(vectors, labels_i32, row_idx, feat_idx):
    T, S = row_idx.shape
    N, D = vectors.shape
    F = feat_idx.shape[1]
    NW = NUM_CORES * NUM_SUBCORES
    assert F == LANES, "feature subset must match SC lane count"
    assert (T * S) % NW == 0
    per_w = (T * S) // NW
    assert NW % T == 0 and S % per_w == 0
    w_per_tree = NW // T
    assert per_w % CHUNK == 0
    nch = per_w // CHUNK

    mesh = plsc.VectorSubcoreMesh(
        core_axis_name="c", subcore_axis_name="s",
        num_cores=NUM_CORES, num_subcores=NUM_SUBCORES)

    @functools.partial(
        pl.kernel,
        out_type=[
            jax.ShapeDtypeStruct((T, S, F), jnp.float32),
            jax.ShapeDtypeStruct((T, S), jnp.int32),
        ],
        mesh=mesh,
        compiler_params=pltpu.CompilerParams(needs_layout_passes=False),
        scratch_types=[
            pltpu.VMEM((per_w,), jnp.int32),
            pltpu.VMEM((F,), jnp.int32),
            pltpu.VMEM((2, CHUNK, 256), jnp.float32),
            pltpu.VMEM((2, CHUNK), jnp.int32),
            pltpu.VMEM((2, CHUNK, 16), jnp.float32),
            pltpu.SemaphoreType.DMA((2,)),
            pltpu.SemaphoreType.DMA((2,)),
            pltpu.SemaphoreType.DMA((2,)),
            pltpu.SemaphoreType.DMA((2,)),
        ],
    )
    def run(vec_hbm, lab_hbm, ri_hbm, fi_hbm, out_hbm, olab_hbm,
            idx_v, feat_v, rbuf, lbuf, obuf, sem_r, sem_l, sem_so, sem_sl):
        wid = lax.axis_index("s") * NUM_CORES + lax.axis_index("c")
        t = wid // w_per_tree
        base = (wid % w_per_tree) * per_w

        pltpu.sync_copy(ri_hbm.at[t, pl.ds(base, per_w)], idx_v)
        pltpu.sync_copy(fi_hbm.at[t], feat_v)
        feat = feat_v[:]

        def start_gathers(c):
            p = c % 2
            idxs = idx_v.at[pl.ds(c * CHUNK, CHUNK)]
            pltpu.async_copy(vec_hbm.at[idxs], rbuf.at[p], sem_r.at[p])
            pltpu.async_copy(lab_hbm.at[idxs], lbuf.at[p], sem_l.at[p])

        def wait_stores(c):
            p = c % 2
            off = base + c * CHUNK
            pltpu.make_async_copy(
                obuf.at[p], out_hbm.at[t, pl.ds(off, CHUNK), :],
                sem_so.at[p]).wait()
            pltpu.make_async_copy(
                lbuf.at[p], olab_hbm.at[t, pl.ds(off, CHUNK)],
                sem_sl.at[p]).wait()

        start_gathers(0)

        def chunk_body(c, carry):
            p = c % 2

            @pl.when(c >= 1)
            def _():
                wait_stores(c - 1)

            @pl.when(c + 1 < nch)
            def _():
                start_gathers(c + 1)

            idxs = idx_v.at[pl.ds(c * CHUNK, CHUNK)]
            pltpu.make_async_copy(vec_hbm.at[idxs], rbuf.at[p], sem_r.at[p]).wait()
            pltpu.make_async_copy(lab_hbm.at[idxs], lbuf.at[p], sem_l.at[p]).wait()

            p16 = jnp.full((LANES,), p, jnp.int32)

            def sub(r, carry2):
                r16 = jnp.full((LANES,), r, jnp.int32)
                obuf[p, r, :] = plsc.load_gather(rbuf, [p16, r16, feat])
                return carry2

            lax.fori_loop(0, CHUNK, sub, None)

            off = base + c * CHUNK
            pltpu.async_copy(
                obuf.at[p], out_hbm.at[t, pl.ds(off, CHUNK), :], sem_so.at[p])
            pltpu.async_copy(
                lbuf.at[p], olab_hbm.at[t, pl.ds(off, CHUNK)], sem_sl.at[p])
            return carry

        lax.fori_loop(0, nch, chunk_body, None)
        wait_stores(nch - 1)

    return run(vectors, labels_i32, row_idx, feat_idx)


def kernel(vectors, labels, row_indices, feat_indices):
    featured, lab = _forest_gather(
        vectors,
        labels.astype(jnp.int32),
        row_indices.astype(jnp.int32),
        feat_indices.astype(jnp.int32),
    )
    return featured, lab.astype(labels.dtype)


# speedup vs baseline: 2.5264x; 1.0031x over previous
import functools

import jax
import jax.numpy as jnp
from jax import lax
from jax.experimental import pallas as pl
from jax.experimental.pallas import tpu as pltpu
from jax.experimental.pallas import tpu_sc as plsc

NUM_CORES = 2
NUM_SUBCORES = 16
LANES = 16
CHUNK = 128


def _forest_gather(vectors, labels_i32, row_idx, feat_idx):
    T, S = row_idx.shape
    N, D = vectors.shape
    F = feat_idx.shape[1]
    NW = NUM_CORES * NUM_SUBCORES
    assert F == LANES, "feature subset must match SC lane count"
    assert (T * S) % NW == 0
    per_w = (T * S) // NW
    assert NW % T == 0 and S % per_w == 0
    w_per_tree = NW // T
    assert per_w % CHUNK == 0
    nch = per_w // CHUNK

    mesh = plsc.VectorSubcoreMesh(
        core_axis_name="c", subcore_axis_name="s",
        num_cores=NUM_CORES, num_subcores=NUM_SUBCORES)

    @functools.partial(
        pl.kernel,
        out_type=[
            jax.ShapeDtypeStruct((T, S, F), jnp.float32),
            jax.ShapeDtypeStruct((T, S), jnp.int32),
        ],
        mesh=mesh,
        compiler_params=pltpu.CompilerParams(
            needs_layout_passes=False,
            disable_bounds_checks=True,
            disable_semaphore_checks=True,
        ),
        scratch_types=[
            pltpu.VMEM((per_w,), jnp.int32),
            pltpu.VMEM((F,), jnp.int32),
            pltpu.VMEM((2, CHUNK, 256), jnp.float32),
            pltpu.VMEM((2, CHUNK), jnp.int32),
            pltpu.VMEM((2, CHUNK, 16), jnp.float32),
            pltpu.SemaphoreType.DMA((2,)),
            pltpu.SemaphoreType.DMA((2,)),
            pltpu.SemaphoreType.DMA((2,)),
            pltpu.SemaphoreType.DMA((2,)),
        ],
    )
    def run(vec_hbm, lab_hbm, ri_hbm, fi_hbm, out_hbm, olab_hbm,
            idx_v, feat_v, rbuf, lbuf, obuf, sem_r, sem_l, sem_so, sem_sl):
        wid = lax.axis_index("s") * NUM_CORES + lax.axis_index("c")
        t = wid // w_per_tree
        base = (wid % w_per_tree) * per_w

        pltpu.sync_copy(ri_hbm.at[t, pl.ds(base, per_w)], idx_v)
        pltpu.sync_copy(fi_hbm.at[t], feat_v)
        feat = feat_v[:]

        def start_gathers(c):
            p = c % 2
            idxs = idx_v.at[pl.ds(c * CHUNK, CHUNK)]
            pltpu.async_copy(vec_hbm.at[idxs], rbuf.at[p], sem_r.at[p])
            pltpu.async_copy(lab_hbm.at[idxs], lbuf.at[p], sem_l.at[p])

        def wait_stores(c):
            p = c % 2
            off = base + c * CHUNK
            pltpu.make_async_copy(
                obuf.at[p], out_hbm.at[t, pl.ds(off, CHUNK), :],
                sem_so.at[p]).wait()
            pltpu.make_async_copy(
                lbuf.at[p], olab_hbm.at[t, pl.ds(off, CHUNK)],
                sem_sl.at[p]).wait()

        start_gathers(0)

        def chunk_body(c, carry):
            p = c % 2

            @pl.when(c >= 1)
            def _():
                wait_stores(c - 1)

            @pl.when(c + 1 < nch)
            def _():
                start_gathers(c + 1)

            idxs = idx_v.at[pl.ds(c * CHUNK, CHUNK)]
            pltpu.make_async_copy(vec_hbm.at[idxs], rbuf.at[p], sem_r.at[p]).wait()
            pltpu.make_async_copy(lab_hbm.at[idxs], lbuf.at[p], sem_l.at[p]).wait()

            p16 = jnp.full((LANES,), p, jnp.int32)

            def sub(r, carry2):
                r16 = jnp.full((LANES,), r, jnp.int32)
                obuf[p, r, :] = plsc.load_gather(rbuf, [p16, r16, feat])
                return carry2

            lax.fori_loop(0, CHUNK, sub, None)

            off = base + c * CHUNK
            pltpu.async_copy(
                obuf.at[p], out_hbm.at[t, pl.ds(off, CHUNK), :], sem_so.at[p])
            pltpu.async_copy(
                lbuf.at[p], olab_hbm.at[t, pl.ds(off, CHUNK)], sem_sl.at[p])
            return carry

        lax.fori_loop(0, nch, chunk_body, None)
        wait_stores(nch - 1)

    return run(vectors, labels_i32, row_idx, feat_idx)


def kernel(vectors, labels, row_indices, feat_indices):
    featured, lab = _forest_gather(
        vectors,
        labels.astype(jnp.int32),
        row_indices.astype(jnp.int32),
        feat_indices.astype(jnp.int32),
    )
    return featured, lab.astype(labels.dtype)
